# baseline (speedup 1.0000x reference)
import functools

import jax
import jax.numpy as jnp
from jax import lax
from jax.experimental import pallas as pl
from jax.experimental.pallas import tpu as pltpu
from jax.experimental.pallas import tpu_sc as plsc

_L = 16
_NW = 32
_CHUNK = 16384


def _lookup_kernel(n_elems):
    per_w = n_elems // _NW
    n_chunks = per_w // _CHUNK
    mesh = plsc.VectorSubcoreMesh(core_axis_name="c", subcore_axis_name="s")

    @functools.partial(
        pl.kernel,
        mesh=mesh,
        out_type=jax.ShapeDtypeStruct((n_elems,), jnp.float32),
        compiler_params=pltpu.CompilerParams(needs_layout_passes=False),
        scratch_types=[
            pltpu.VMEM((256,), jnp.float32),
            pltpu.VMEM((_CHUNK,), jnp.int32),
            pltpu.VMEM((_CHUNK,), jnp.int32),
            pltpu.VMEM((_CHUNK,), jnp.float32),
            pltpu.VMEM((_CHUNK,), jnp.float32),
            pltpu.SemaphoreType.DMA,
            pltpu.SemaphoreType.DMA,
            pltpu.SemaphoreType.DMA,
            pltpu.SemaphoreType.DMA,
        ],
    )
    def k(tbl_hbm, idx_hbm, out_hbm, tbl_v, idx_v0, idx_v1, out_v0, out_v1,
          in_s0, in_s1, out_s0, out_s1):
        wid = lax.axis_index("s") * 2 + lax.axis_index("c")
        base = wid * per_w
        pltpu.sync_copy(tbl_hbm, tbl_v)
        idx_bufs = [idx_v0, idx_v1]
        out_bufs = [out_v0, out_v1]
        in_sems = [in_s0, in_s1]
        out_sems = [out_s0, out_s1]

        def in_copy(kk, b):
            return pltpu.make_async_copy(
                idx_hbm.at[pl.ds(base + kk * _CHUNK, _CHUNK)],
                idx_bufs[b],
                in_sems[b],
            )

        def out_copy(kk, b):
            return pltpu.make_async_copy(
                out_bufs[b],
                out_hbm.at[pl.ds(base + kk * _CHUNK, _CHUNK)],
                out_sems[b],
            )

        in_copy(0, 0).start()
        for kk in range(n_chunks):
            b = kk & 1
            if kk + 1 < n_chunks:
                in_copy(kk + 1, 1 - b).start()
            in_copy(kk, b).wait()
            if kk >= 2:
                out_copy(kk - 2, b).wait()
            idx_b = idx_bufs[b]
            out_b = out_bufs[b]

            @plsc.parallel_loop(0, _CHUNK, step=_L, unroll=16)
            def inner(s):
                iv = idx_b[pl.ds(s, _L)]
                out_b[pl.ds(s, _L)] = plsc.load_gather(tbl_v, [iv])

            out_copy(kk, b).start()
        out_copy(n_chunks - 2, (n_chunks - 2) & 1).wait()
        out_copy(n_chunks - 1, (n_chunks - 1) & 1).wait()

    return k


@jax.jit
def kernel(class_map, img):
    n = img.size
    flat = img.reshape(n)
    out = _lookup_kernel(n)(class_map, flat)
    return out.reshape(img.shape)


# speedup vs baseline: 1.0155x; 1.0155x over previous
import functools

import jax
import jax.numpy as jnp
from jax import lax
from jax.experimental import pallas as pl
from jax.experimental.pallas import tpu as pltpu
from jax.experimental.pallas import tpu_sc as plsc

_L = 16
_NW = 32
_CHUNK = 16384


def _lookup_kernel(n_elems):
    per_w = n_elems // _NW
    n_chunks = per_w // _CHUNK
    mesh = plsc.VectorSubcoreMesh(core_axis_name="c", subcore_axis_name="s")

    @functools.partial(
        pl.kernel,
        mesh=mesh,
        out_type=jax.ShapeDtypeStruct((n_elems,), jnp.float32),
        compiler_params=pltpu.CompilerParams(needs_layout_passes=False),
        scratch_types=[
            pltpu.VMEM((256,), jnp.float32),
            pltpu.VMEM((_CHUNK,), jnp.int32),
            pltpu.VMEM((_CHUNK,), jnp.int32),
            pltpu.VMEM((_CHUNK,), jnp.float32),
            pltpu.VMEM((_CHUNK,), jnp.float32),
            pltpu.SemaphoreType.DMA,
            pltpu.SemaphoreType.DMA,
            pltpu.SemaphoreType.DMA,
            pltpu.SemaphoreType.DMA,
        ],
    )
    def k(tbl_hbm, idx_hbm, out_hbm, tbl_v, idx_v0, idx_v1, out_v0, out_v1,
          in_s0, in_s1, out_s0, out_s1):
        wid = lax.axis_index("s") * 2 + lax.axis_index("c")
        base = wid * per_w
        pltpu.sync_copy(tbl_hbm, tbl_v)
        idx_bufs = [idx_v0, idx_v1]
        out_bufs = [out_v0, out_v1]
        in_sems = [in_s0, in_s1]
        out_sems = [out_s0, out_s1]

        def in_copy(kk, b):
            return pltpu.make_async_copy(
                idx_hbm.at[pl.ds(base + kk * _CHUNK, _CHUNK)],
                idx_bufs[b],
                in_sems[b],
            )

        def out_copy(kk, b):
            return pltpu.make_async_copy(
                out_bufs[b],
                out_hbm.at[pl.ds(base + kk * _CHUNK, _CHUNK)],
                out_sems[b],
            )

        in_copy(0, 0).start()
        for kk in range(n_chunks):
            b = kk & 1
            if kk + 1 < n_chunks:
                in_copy(kk + 1, 1 - b).start()
            in_copy(kk, b).wait()
            if kk >= 2:
                out_copy(kk - 2, b).wait()
            idx_b = idx_bufs[b]
            out_b = out_bufs[b]

            @plsc.parallel_loop(0, _CHUNK, step=_L, unroll=8)
            def inner(s):
                iv = idx_b[pl.ds(s, _L)]
                out_b[pl.ds(s, _L)] = plsc.load_gather(tbl_v, [iv])

            out_copy(kk, b).start()
        out_copy(n_chunks - 2, (n_chunks - 2) & 1).wait()
        out_copy(n_chunks - 1, (n_chunks - 1) & 1).wait()

    return k


@jax.jit
def kernel(class_map, img):
    n = img.size
    flat = img.reshape(n)
    out = _lookup_kernel(n)(class_map, flat)
    return out.reshape(img.shape)


# speedup vs baseline: 2.5319x; 2.4934x over previous
import functools

import jax
import jax.numpy as jnp
from jax import lax
from jax.experimental import pallas as pl
from jax.experimental.pallas import tpu as pltpu
from jax.experimental.pallas import tpu_sc as plsc

_L = 16
_NW = 32
_ROWS = 32
_COLS = 512


def _lookup_kernel(shape):
    n_imgs, n_rows, n_cols = shape
    assert n_cols == _COLS and n_rows % _ROWS == 0 and (2 * _NW) == n_imgs
    slabs_per_img = n_rows // _ROWS
    n_chunks = 2 * slabs_per_img
    mesh = plsc.VectorSubcoreMesh(core_axis_name="c", subcore_axis_name="s")

    @functools.partial(
        pl.kernel,
        mesh=mesh,
        out_type=jax.ShapeDtypeStruct(shape, jnp.float32),
        compiler_params=pltpu.CompilerParams(needs_layout_passes=False),
        scratch_types=[
            pltpu.VMEM((256,), jnp.float32),
            pltpu.VMEM((_ROWS, _COLS), jnp.int32),
            pltpu.VMEM((_ROWS, _COLS), jnp.int32),
            pltpu.VMEM((_ROWS, _COLS), jnp.float32),
            pltpu.VMEM((_ROWS, _COLS), jnp.float32),
            pltpu.SemaphoreType.DMA,
            pltpu.SemaphoreType.DMA,
            pltpu.SemaphoreType.DMA,
            pltpu.SemaphoreType.DMA,
        ],
    )
    def k(tbl_hbm, idx_hbm, out_hbm, tbl_v, idx_v0, idx_v1, out_v0, out_v1,
          in_s0, in_s1, out_s0, out_s1):
        wid = lax.axis_index("s") * 2 + lax.axis_index("c")
        img0 = wid * 2
        pltpu.sync_copy(tbl_hbm, tbl_v)
        idx_bufs = [idx_v0, idx_v1]
        out_bufs = [out_v0, out_v1]
        in_sems = [in_s0, in_s1]
        out_sems = [out_s0, out_s1]

        def slab(kk):
            imgi = img0 + (kk // slabs_per_img)
            r0 = (kk % slabs_per_img) * _ROWS
            return imgi, r0

        def in_copy(kk, b):
            imgi, r0 = slab(kk)
            return pltpu.make_async_copy(
                idx_hbm.at[imgi, pl.ds(r0, _ROWS), :],
                idx_bufs[b],
                in_sems[b],
            )

        def out_copy(kk, b):
            imgi, r0 = slab(kk)
            return pltpu.make_async_copy(
                out_bufs[b],
                out_hbm.at[imgi, pl.ds(r0, _ROWS), :],
                out_sems[b],
            )

        in_copy(0, 0).start()
        for kk in range(n_chunks):
            b = kk & 1
            if kk + 1 < n_chunks:
                in_copy(kk + 1, 1 - b).start()
            in_copy(kk, b).wait()
            if kk >= 2:
                out_copy(kk - 2, b).wait()
            idx_b = idx_bufs[b]
            out_b = out_bufs[b]

            @plsc.parallel_loop(0, _ROWS * _COLS, step=_L, unroll=8)
            def inner(s):
                r = s // _COLS
                c = s % _COLS
                iv = idx_b[r, pl.ds(c, _L)]
                out_b[r, pl.ds(c, _L)] = plsc.load_gather(tbl_v, [iv])

            out_copy(kk, b).start()
        out_copy(n_chunks - 2, (n_chunks - 2) & 1).wait()
        out_copy(n_chunks - 1, (n_chunks - 1) & 1).wait()

    return k


@jax.jit
def kernel(class_map, img):
    return _lookup_kernel(img.shape)(class_map, img)
